# baseline (speedup 1.0000x reference)
import jax
import jax.numpy as jnp
from jax import lax
from jax.experimental import pallas as pl
from jax.experimental.pallas import tpu as pltpu
from jax.experimental.pallas import tpu_sc as plsc

VOCAB = 100000
EMB = 128
SEQ = 4096
TOK = 50
TOKP = 56

_info = plsc.get_sparse_core_info()
NC, NS = _info.num_cores, _info.num_subcores
NW = NC * NS

B = SEQ * TOKP
B_PER_W = B // NW
CH = 128
N_CHUNKS = B_PER_W // CH
NBUF = 4
NG = N_CHUNKS // NBUF


def _body(x_hbm, table_hbm, out_hbm, idx_v, *rest):
    rows = rest[:NBUF]
    gsems = rest[NBUF:2 * NBUF]
    ssems = rest[2 * NBUF:3 * NBUF]
    wid = lax.axis_index("s") * NC + lax.axis_index("c")
    base = wid * B_PER_W
    pltpu.sync_copy(x_hbm.at[wid], idx_v)

    def gather_wait(b):
        pltpu.make_async_copy(table_hbm.at[pl.ds(0, CH)], rows[b],
                              gsems[b]).wait()

    def store_wait(b):
        pltpu.make_async_copy(rows[b], out_hbm.at[pl.ds(0, CH)],
                              ssems[b]).wait()

    for b in range(NBUF):
        pltpu.async_copy(table_hbm.at[idx_v.at[b]], rows[b], gsems[b])

    def grp(t, carry):
        for b in range(NBUF):
            i = t * NBUF + b
            gather_wait(b)
            pltpu.async_copy(rows[b], out_hbm.at[pl.ds(base + i * CH, CH)],
                             ssems[b])

        @pl.when(t < NG - 1)
        def _prefetch():
            for b in range(NBUF):
                store_wait(b)
                pltpu.async_copy(table_hbm.at[idx_v.at[(t + 1) * NBUF + b]],
                                 rows[b], gsems[b])

        return carry

    lax.fori_loop(0, NG, grp, 0)
    for b in range(NBUF):
        store_wait(b)


@jax.jit
def _lookup(x_flat, table):
    mesh = plsc.VectorSubcoreMesh(core_axis_name="c", subcore_axis_name="s")
    return pl.kernel(
        _body,
        out_type=jax.ShapeDtypeStruct((B, EMB), jnp.float32),
        mesh=mesh,
        scratch_types=(
            [pltpu.VMEM((N_CHUNKS, CH), jnp.int32)]
            + [pltpu.VMEM((CH, EMB), jnp.float32) for _ in range(NBUF)]
            + [pltpu.SemaphoreType.DMA for _ in range(2 * NBUF)]
        ),
    )(x_flat, table)


def kernel(x, table):
    x_pad = jnp.pad(x.astype(jnp.int32), ((0, 0), (0, TOKP - TOK)))
    x_flat = x_pad.reshape(NW, N_CHUNKS, CH)
    out = _lookup(x_flat, table)
    return out.reshape(SEQ, TOKP, EMB)[:, :TOK, :]


# speedup vs baseline: 6.2066x; 6.2066x over previous
import jax
import jax.numpy as jnp
from jax import lax
from jax.experimental import pallas as pl
from jax.experimental.pallas import tpu as pltpu
from jax.experimental.pallas import tpu_sc as plsc

VOCAB = 100000
EMB = 128
SEQ = 4096
TOK = 50
TOKP = 56

_info = plsc.get_sparse_core_info()
NC, NS = _info.num_cores, _info.num_subcores
NW = NC * NS

B = SEQ * TOKP
B_PER_W = B // NW
CH = 128
N_CHUNKS = B_PER_W // CH
NBUF = 4
NG = N_CHUNKS // NBUF


def _body(x_hbm, table_hbm, out_hbm, idx_v, *rest):
    rows = rest[:NBUF]
    gsems = rest[NBUF:2 * NBUF]
    ssems = rest[2 * NBUF:3 * NBUF]
    wid = lax.axis_index("s") * NC + lax.axis_index("c")
    base = wid * B_PER_W
    pltpu.sync_copy(x_hbm.at[wid], idx_v)

    def gather_wait(b):
        pltpu.make_async_copy(table_hbm.at[pl.ds(0, CH)], rows[b],
                              gsems[b]).wait()

    def store_wait(b):
        pltpu.make_async_copy(rows[b], out_hbm.at[pl.ds(0, CH)],
                              ssems[b]).wait()

    for b in range(NBUF):
        pltpu.async_copy(table_hbm.at[idx_v.at[b]], rows[b], gsems[b])

    def grp(t, carry):
        for b in range(NBUF):
            i = t * NBUF + b
            gather_wait(b)
            pltpu.async_copy(rows[b], out_hbm.at[pl.ds(base + i * CH, CH)],
                             ssems[b])

        @pl.when(t < NG - 1)
        def _prefetch():
            for b in range(NBUF):
                store_wait(b)
                pltpu.async_copy(table_hbm.at[idx_v.at[(t + 1) * NBUF + b]],
                                 rows[b], gsems[b])

        return carry

    lax.fori_loop(0, NG, grp, 0)
    for b in range(NBUF):
        store_wait(b)


@jax.jit
def _lookup(x_flat, table):
    mesh = plsc.VectorSubcoreMesh(core_axis_name="c", subcore_axis_name="s")
    return pl.kernel(
        _body,
        out_type=jax.ShapeDtypeStruct((B, EMB), jnp.float32),
        mesh=mesh,
        scratch_types=(
            [pltpu.VMEM((N_CHUNKS, CH), jnp.int32)]
            + [pltpu.VMEM((CH, EMB), jnp.float32) for _ in range(NBUF)]
            + [pltpu.SemaphoreType.DMA for _ in range(2 * NBUF)]
        ),
    )(x_flat, table)


def kernel(x, table):
    x_pad = jnp.pad(x.astype(jnp.int32), ((0, 0), (0, TOKP - TOK)),
                    mode="edge")
    x_flat = x_pad.reshape(NW, N_CHUNKS, CH)
    out = _lookup(x_flat, table)
    return out.reshape(SEQ, TOKP, EMB)[:, :TOK, :]


# speedup vs baseline: 7.2112x; 1.1619x over previous
import jax
import jax.numpy as jnp
from jax import lax
from jax.experimental import pallas as pl
from jax.experimental.pallas import tpu as pltpu
from jax.experimental.pallas import tpu_sc as plsc

VOCAB = 100000
EMB = 128
SEQ = 4096
TOK = 50
TOKP = 56

_info = plsc.get_sparse_core_info()
NC, NS = _info.num_cores, _info.num_subcores
NW = NC * NS

NSEQ_W = SEQ // NW
CH_SEQ = 2
NBUF = 4
NGRP = NSEQ_W // CH_SEQ
NT = NGRP // NBUF


def _body(x_hbm, table_hbm, out_hbm, idx_v, *rest):
    rows = rest[:NBUF]
    gsems = rest[NBUF:2 * NBUF]
    ssems = rest[2 * NBUF:3 * NBUF]
    wid = lax.axis_index("s") * NC + lax.axis_index("c")
    wbase = wid * NSEQ_W
    pltpu.sync_copy(x_hbm.at[pl.ds(wbase, NSEQ_W)], idx_v)

    def fire_gather(g, b):
        for j in range(CH_SEQ):
            pltpu.async_copy(table_hbm.at[idx_v.at[g * CH_SEQ + j]],
                             rows[b].at[j], gsems[b])

    def gather_wait(b):
        for j in range(CH_SEQ):
            pltpu.make_async_copy(table_hbm.at[pl.ds(0, TOKP)],
                                  rows[b].at[j], gsems[b]).wait()

    def fire_store(g, b):
        pltpu.async_copy(rows[b].at[:, pl.ds(0, TOK), :],
                         out_hbm.at[pl.ds(wbase + g * CH_SEQ, CH_SEQ)],
                         ssems[b])

    def store_wait(b):
        pltpu.make_async_copy(rows[b].at[:, pl.ds(0, TOK), :],
                              out_hbm.at[pl.ds(0, CH_SEQ)], ssems[b]).wait()

    for b in range(NBUF):
        fire_gather(b, b)

    def grp(t, carry):
        for b in range(NBUF):
            gather_wait(b)
            fire_store(t * NBUF + b, b)

        @pl.when(t < NT - 1)
        def _prefetch():
            for b in range(NBUF):
                store_wait(b)
                fire_gather((t + 1) * NBUF + b, b)

        return carry

    lax.fori_loop(0, NT, grp, 0)
    for b in range(NBUF):
        store_wait(b)


@jax.jit
def _lookup(x_pad, table):
    mesh = plsc.VectorSubcoreMesh(core_axis_name="c", subcore_axis_name="s")
    return pl.kernel(
        _body,
        out_type=jax.ShapeDtypeStruct((SEQ, TOK, EMB), jnp.float32),
        mesh=mesh,
        compiler_params=pltpu.CompilerParams(use_tc_tiling_on_sc=True),
        scratch_types=(
            [pltpu.VMEM((NSEQ_W, TOKP), jnp.int32)]
            + [pltpu.VMEM((CH_SEQ, TOKP, EMB), jnp.float32)
               for _ in range(NBUF)]
            + [pltpu.SemaphoreType.DMA for _ in range(2 * NBUF)]
        ),
    )(x_pad, table)


def kernel(x, table):
    x_pad = jnp.pad(x.astype(jnp.int32), ((0, 0), (0, TOKP - TOK)),
                    mode="edge")
    return _lookup(x_pad, table)


# speedup vs baseline: 13.5545x; 1.8796x over previous
import jax
import jax.numpy as jnp
from jax import lax
from jax.experimental import pallas as pl
from jax.experimental.pallas import tpu as pltpu
from jax.experimental.pallas import tpu_sc as plsc

VOCAB = 100000
EMB = 128
SEQ = 4096
TOK = 50

_info = plsc.get_sparse_core_info()
NC, NS = _info.num_cores, _info.num_subcores
NW = NC * NS

B = SEQ * TOK
B_PER_W = B // NW
CH = 64
N_CHUNKS = B_PER_W // CH
NBUF = 10
NG = N_CHUNKS // NBUF


def _body(x_hbm, table_hbm, out_hbm, idx_v, *rest):
    rows = rest[:NBUF]
    gsems = rest[NBUF:2 * NBUF]
    ssems = rest[2 * NBUF:3 * NBUF]
    wid = lax.axis_index("s") * NC + lax.axis_index("c")
    base = wid * B_PER_W
    pltpu.sync_copy(x_hbm.at[wid], idx_v)

    def gather_wait(b):
        pltpu.make_async_copy(table_hbm.at[pl.ds(0, CH)], rows[b],
                              gsems[b]).wait()

    def store_wait(b):
        pltpu.make_async_copy(rows[b], out_hbm.at[pl.ds(0, CH)],
                              ssems[b]).wait()

    for b in range(NBUF):
        pltpu.async_copy(table_hbm.at[idx_v.at[b]], rows[b], gsems[b])

    def grp(t, carry):
        for b in range(NBUF):
            i = t * NBUF + b
            gather_wait(b)
            pltpu.async_copy(rows[b], out_hbm.at[pl.ds(base + i * CH, CH)],
                             ssems[b])

        @pl.when(t < NG - 1)
        def _prefetch():
            for b in range(NBUF):
                store_wait(b)
                pltpu.async_copy(table_hbm.at[idx_v.at[(t + 1) * NBUF + b]],
                                 rows[b], gsems[b])

        return carry

    lax.fori_loop(0, NG, grp, 0)
    for b in range(NBUF):
        store_wait(b)


@jax.jit
def _lookup(x_flat, table):
    mesh = plsc.VectorSubcoreMesh(core_axis_name="c", subcore_axis_name="s")
    return pl.kernel(
        _body,
        out_type=jax.ShapeDtypeStruct((B, EMB), jnp.float32),
        mesh=mesh,
        scratch_types=(
            [pltpu.VMEM((N_CHUNKS, CH), jnp.int32)]
            + [pltpu.VMEM((CH, EMB), jnp.float32) for _ in range(NBUF)]
            + [pltpu.SemaphoreType.DMA for _ in range(2 * NBUF)]
        ),
    )(x_flat, table)


def kernel(x, table):
    x_t = x.astype(jnp.int32).T.reshape(NW, N_CHUNKS, CH)
    out = _lookup(x_t, table)
    return out.reshape(TOK, SEQ, EMB).transpose(1, 0, 2)
